# baseline (device time: 85391 ns/iter reference)
import jax
import jax.numpy as jnp
from jax import lax
from jax.experimental import pallas as pl
from jax.experimental.pallas import tpu as pltpu

B, SQ, H, D = 4, 256, 16, 64
HD = H * D
ROWS = B * SQ
SCALE = D ** -0.5

C = 16
CH = ROWS // C


def _comm_body(k_ref, v_ref, kr_ref, vr_ref, krbuf, vrbuf,
               s1, r1, s2, r2, ybar, xbar):
    my_x = lax.axis_index("x")
    my_y = lax.axis_index("y")
    ynbr = (my_x, 1 - my_y)
    xnbr = (1 - my_x, my_y)

    pl.semaphore_signal(ybar, inc=1, device_id=ynbr,
                        device_id_type=pl.DeviceIdType.MESH)
    pl.semaphore_signal(xbar, inc=1, device_id=xnbr,
                        device_id_type=pl.DeviceIdType.MESH)
    pl.semaphore_wait(ybar, 1)
    pl.semaphore_wait(xbar, 1)

    def run(src_ref, recv1, recv2, out1, out2):
        mk = lambda **kw: pltpu.make_async_remote_copy(
            device_id_type=pl.DeviceIdType.MESH, **kw)
        ch = lambda ref, i: ref.at[pl.ds(i * CH, CH)]
        p1 = [mk(src_ref=ch(src_ref, i), dst_ref=ch(recv1, i),
                 send_sem=s1.at[i], recv_sem=r1.at[i], device_id=ynbr)
              for i in range(C)]
        fwd = [mk(src_ref=ch(recv1, i), dst_ref=ch(recv1, i),
                  send_sem=s2.at[i], recv_sem=r2.at[i], device_id=xnbr)
               for i in range(C)]
        p2w = [mk(src_ref=ch(src_ref, i), dst_ref=ch(recv2, i),
                  send_sem=s1.at[i], recv_sem=r2.at[i], device_id=xnbr)
               for i in range(C)]
        for d in p1:
            d.start()
        for i in range(C):
            p1[i].wait_recv()
            fwd[i].start()
            out1[pl.ds(i * CH, CH), :] = recv1[pl.ds(i * CH, CH), :]
        for i in range(C):
            p2w[i].wait_recv()
            out2[pl.ds(i * CH, CH), :] = recv2[pl.ds(i * CH, CH), :]
        for i in range(C):
            p1[i].wait_send()
            fwd[i].wait_send()

    @pl.when(my_x == 0)
    def _():
        run(k_ref, krbuf, vrbuf, kr_ref, vr_ref)

    @pl.when(my_x == 1)
    def _():
        run(v_ref, vrbuf, krbuf, vr_ref, kr_ref)


def _exchange(kb, vb):
    return pl.pallas_call(
        _comm_body,
        out_shape=(
            jax.ShapeDtypeStruct((ROWS, HD), jnp.bfloat16),
            jax.ShapeDtypeStruct((ROWS, HD), jnp.bfloat16),
        ),
        in_specs=[pl.BlockSpec(memory_space=pltpu.VMEM)] * 2,
        out_specs=(pl.BlockSpec(memory_space=pltpu.VMEM),) * 2,
        scratch_shapes=[
            pltpu.VMEM((ROWS, HD), jnp.bfloat16),
            pltpu.VMEM((ROWS, HD), jnp.bfloat16),
            pltpu.SemaphoreType.DMA((C,)),
            pltpu.SemaphoreType.DMA((C,)),
            pltpu.SemaphoreType.DMA((C,)),
            pltpu.SemaphoreType.DMA((C,)),
            pltpu.SemaphoreType.REGULAR,
            pltpu.SemaphoreType.REGULAR,
        ],
    )(kb, vb)


def _one_head(q, kcat, vcat):
    nt = (((1,), (1,)), ((), ()))
    nn = (((1,), (0,)), ((), ()))
    s = lax.dot_general(q, kcat, nt, preferred_element_type=jnp.float32) * SCALE
    m = jnp.max(s, axis=1, keepdims=True)
    p = jnp.exp(s - m)
    denom = jnp.sum(p, axis=1, keepdims=True)
    o = lax.dot_general(p.astype(jnp.bfloat16), vcat, nn,
                        preferred_element_type=jnp.float32)
    return o / denom


def _attn_body(q_ref, kl_ref, vl_ref, kr_ref, vr_ref, o_ref):
    q = q_ref[0].astype(jnp.bfloat16)
    kl = kl_ref[0].astype(jnp.bfloat16)
    vl = vl_ref[0].astype(jnp.bfloat16)
    kr = kr_ref[...]
    vr = vr_ref[...]
    outs = []
    for h in range(H):
        hs = slice(h * D, (h + 1) * D)
        kcat = jnp.concatenate([kl[:, hs], kr[:, hs]], axis=0)
        vcat = jnp.concatenate([vl[:, hs], vr[:, hs]], axis=0)
        outs.append(_one_head(q[:, hs], kcat, vcat))
    o_ref[0] = jnp.concatenate(outs, axis=1)


def kernel(Q, K, V):
    kb = K.astype(jnp.bfloat16).reshape(ROWS, HD)
    vb = V.astype(jnp.bfloat16).reshape(ROWS, HD)
    k_rem, v_rem = _exchange(kb, vb)

    q3 = Q.reshape(B, SQ, HD)
    kl3 = kb.reshape(B, SQ, HD)
    vl3 = vb.reshape(B, SQ, HD)
    blk3 = lambda: pl.BlockSpec((1, SQ, HD), lambda b: (b, 0, 0))
    blk2 = lambda: pl.BlockSpec((SQ, HD), lambda b: (b, 0))
    out = pl.pallas_call(
        _attn_body,
        grid=(B,),
        in_specs=[blk3(), blk3(), blk3(), blk2(), blk2()],
        out_specs=blk3(),
        out_shape=jax.ShapeDtypeStruct((B, SQ, HD), jnp.float32),
        compiler_params=pltpu.CompilerParams(
            dimension_semantics=("arbitrary",)),
    )(q3, kl3, vl3, k_rem, v_rem)
    return out.reshape(B, SQ, H, D)


# device time: 79280 ns/iter; 1.0771x vs baseline; 1.0771x over previous
import jax
import jax.numpy as jnp
from jax import lax
from jax.experimental import pallas as pl
from jax.experimental.pallas import tpu as pltpu

B, SQ, H, D = 4, 256, 16, 64
ROWS = B * H * D
SCALE = D ** -0.5

C = 16
CH = ROWS // C


def _comm_body(k_ref, v_ref, kr_ref, vr_ref, sendbuf, krbuf, vrbuf,
               s1, r1, s2, r2, ybar, xbar):
    my_x = lax.axis_index("x")
    my_y = lax.axis_index("y")
    ynbr = (my_x, 1 - my_y)
    xnbr = (1 - my_x, my_y)

    pl.semaphore_signal(ybar, inc=1, device_id=ynbr,
                        device_id_type=pl.DeviceIdType.MESH)
    pl.semaphore_signal(xbar, inc=1, device_id=xnbr,
                        device_id_type=pl.DeviceIdType.MESH)
    pl.semaphore_wait(ybar, 1)
    pl.semaphore_wait(xbar, 1)

    def run(src_ref, recv1, recv2, out1, out2):
        mk = lambda **kw: pltpu.make_async_remote_copy(
            device_id_type=pl.DeviceIdType.MESH, **kw)
        ch = lambda ref, i: ref.at[pl.ds(i * CH, CH)]
        p1 = [mk(src_ref=ch(sendbuf, i), dst_ref=ch(recv1, i),
                 send_sem=s1.at[i], recv_sem=r1.at[i], device_id=ynbr)
              for i in range(C)]
        fwd = [mk(src_ref=ch(recv1, i), dst_ref=ch(recv1, i),
                  send_sem=s2.at[i], recv_sem=r2.at[i], device_id=xnbr)
               for i in range(C)]
        p2w = [mk(src_ref=ch(sendbuf, i), dst_ref=ch(recv2, i),
                  send_sem=s1.at[i], recv_sem=r2.at[i], device_id=xnbr)
               for i in range(C)]
        for i in range(C):
            sendbuf[pl.ds(i * CH, CH), :] = (
                src_ref[pl.ds(i * CH, CH), :].astype(jnp.bfloat16))
            p1[i].start()
        for i in range(C):
            p1[i].wait_recv()
            fwd[i].start()
            out1[pl.ds(i * CH, CH), :] = recv1[pl.ds(i * CH, CH), :]
        for i in range(C):
            p2w[i].wait_recv()
            out2[pl.ds(i * CH, CH), :] = recv2[pl.ds(i * CH, CH), :]
        for i in range(C):
            p1[i].wait_send()
            fwd[i].wait_send()

    @pl.when(my_x == 0)
    def _():
        run(k_ref, krbuf, vrbuf, kr_ref, vr_ref)

    @pl.when(my_x == 1)
    def _():
        run(v_ref, vrbuf, krbuf, vr_ref, kr_ref)


def _exchange(ktf, vtf):
    return pl.pallas_call(
        _comm_body,
        out_shape=(
            jax.ShapeDtypeStruct((ROWS, SQ), jnp.bfloat16),
            jax.ShapeDtypeStruct((ROWS, SQ), jnp.bfloat16),
        ),
        in_specs=[pl.BlockSpec(memory_space=pltpu.VMEM)] * 2,
        out_specs=(pl.BlockSpec(memory_space=pltpu.VMEM),) * 2,
        scratch_shapes=[
            pltpu.VMEM((ROWS, SQ), jnp.bfloat16),
            pltpu.VMEM((ROWS, SQ), jnp.bfloat16),
            pltpu.VMEM((ROWS, SQ), jnp.bfloat16),
            pltpu.SemaphoreType.DMA((C,)),
            pltpu.SemaphoreType.DMA((C,)),
            pltpu.SemaphoreType.DMA((C,)),
            pltpu.SemaphoreType.DMA((C,)),
            pltpu.SemaphoreType.REGULAR,
            pltpu.SemaphoreType.REGULAR,
        ],
    )(ktf, vtf)


def _attn_body(q_ref, kl_ref, vl_ref, kr_ref, vr_ref, o_ref):
    qT = q_ref[0, 0].astype(jnp.bfloat16)
    klT = kl_ref[0, 0].astype(jnp.bfloat16)
    vlT = vl_ref[0, 0].astype(jnp.bfloat16)
    krT = kr_ref[0, 0]
    vrT = vr_ref[0, 0]

    tn = (((0,), (0,)), ((), ()))
    kk = (((1,), (1,)), ((), ()))
    sl = lax.dot_general(qT, klT, tn, preferred_element_type=jnp.float32) * SCALE
    sr = lax.dot_general(qT, krT, tn, preferred_element_type=jnp.float32) * SCALE
    m = jnp.maximum(jnp.max(sl, axis=1, keepdims=True),
                    jnp.max(sr, axis=1, keepdims=True))
    pl_ = jnp.exp(sl - m)
    pr = jnp.exp(sr - m)
    denom = jnp.sum(pl_, axis=1, keepdims=True) + jnp.sum(pr, axis=1, keepdims=True)
    pl_ = (pl_ / denom).astype(jnp.bfloat16)
    pr = (pr / denom).astype(jnp.bfloat16)
    ol = lax.dot_general(vlT, pl_, kk, preferred_element_type=jnp.float32)
    orr = lax.dot_general(vrT, pr, kk, preferred_element_type=jnp.float32)
    o_ref[0, 0] = ol + orr


def kernel(Q, K, V):
    QT = jnp.transpose(Q, (0, 2, 3, 1))
    KT = jnp.transpose(K, (0, 2, 3, 1))
    VT = jnp.transpose(V, (0, 2, 3, 1))

    k_rem, v_rem = _exchange(KT.reshape(ROWS, SQ), VT.reshape(ROWS, SQ))
    kr4 = k_rem.reshape(B, H, D, SQ)
    vr4 = v_rem.reshape(B, H, D, SQ)

    blk = lambda: pl.BlockSpec((1, 1, D, SQ), lambda b, h: (b, h, 0, 0))
    out = pl.pallas_call(
        _attn_body,
        grid=(B, H),
        in_specs=[blk() for _ in range(5)],
        out_specs=blk(),
        out_shape=jax.ShapeDtypeStruct((B, H, D, SQ), jnp.float32),
        compiler_params=pltpu.CompilerParams(
            dimension_semantics=("arbitrary", "arbitrary")),
    )(QT, KT, VT, kr4, vr4)
    return out.transpose(0, 3, 1, 2)


# device time: 57311 ns/iter; 1.4900x vs baseline; 1.3833x over previous
import jax
import jax.numpy as jnp
from jax import lax
from jax.experimental import pallas as pl
from jax.experimental.pallas import tpu as pltpu

B, SQ, H, D = 4, 256, 16, 64
ROWS = B * H * D
SCALE = D ** -0.5

C = 16
CH = ROWS // C


def _comm_body(k_ref, v_ref, kr_ref, vr_ref, sendbuf, krbuf, vrbuf,
               s1, r1, s2, r2, ybar, xbar):
    my_x = lax.axis_index("x")
    my_y = lax.axis_index("y")
    ynbr = (my_x, 1 - my_y)
    xnbr = (1 - my_x, my_y)

    pl.semaphore_signal(ybar, inc=1, device_id=ynbr,
                        device_id_type=pl.DeviceIdType.MESH)
    pl.semaphore_signal(xbar, inc=1, device_id=xnbr,
                        device_id_type=pl.DeviceIdType.MESH)
    pl.semaphore_wait(ybar, 1)
    pl.semaphore_wait(xbar, 1)

    def run(src_ref, recv1, recv2, out1, out2):
        mk = lambda **kw: pltpu.make_async_remote_copy(
            device_id_type=pl.DeviceIdType.MESH, **kw)
        ch = lambda ref, i: ref.at[pl.ds(i * CH, CH)]
        p1 = [mk(src_ref=ch(sendbuf, i), dst_ref=ch(recv1, i),
                 send_sem=s1.at[i], recv_sem=r1.at[i], device_id=ynbr)
              for i in range(C)]
        fwd = [mk(src_ref=ch(recv1, i), dst_ref=ch(recv1, i),
                  send_sem=s2.at[i], recv_sem=r2.at[i], device_id=xnbr)
               for i in range(C)]
        p2w = [mk(src_ref=ch(sendbuf, i), dst_ref=ch(recv2, i),
                  send_sem=s1.at[i], recv_sem=r2.at[i], device_id=xnbr)
               for i in range(C)]
        for i in range(C):
            sendbuf[pl.ds(i * CH, CH), :] = (
                src_ref[pl.ds(i * CH, CH), :].astype(jnp.bfloat16))
            p1[i].start()
        for i in range(C):
            p1[i].wait_recv()
            fwd[i].start()
            out1[pl.ds(i * CH, CH), :] = recv1[pl.ds(i * CH, CH), :]
        for i in range(C):
            p2w[i].wait_recv()
            out2[pl.ds(i * CH, CH), :] = recv2[pl.ds(i * CH, CH), :]
        for i in range(C):
            p1[i].wait_send()
            fwd[i].wait_send()

    @pl.when(my_x == 0)
    def _():
        run(k_ref, krbuf, vrbuf, kr_ref, vr_ref)

    @pl.when(my_x == 1)
    def _():
        run(v_ref, vrbuf, krbuf, vr_ref, kr_ref)


def _exchange(ktf, vtf):
    return pl.pallas_call(
        _comm_body,
        out_shape=(
            jax.ShapeDtypeStruct((ROWS, SQ), jnp.bfloat16),
            jax.ShapeDtypeStruct((ROWS, SQ), jnp.bfloat16),
        ),
        in_specs=[pl.BlockSpec(memory_space=pltpu.VMEM)] * 2,
        out_specs=(pl.BlockSpec(memory_space=pltpu.VMEM),) * 2,
        scratch_shapes=[
            pltpu.VMEM((ROWS, SQ), jnp.bfloat16),
            pltpu.VMEM((ROWS, SQ), jnp.bfloat16),
            pltpu.VMEM((ROWS, SQ), jnp.bfloat16),
            pltpu.SemaphoreType.DMA((C,)),
            pltpu.SemaphoreType.DMA((C,)),
            pltpu.SemaphoreType.DMA((C,)),
            pltpu.SemaphoreType.DMA((C,)),
            pltpu.SemaphoreType.REGULAR,
            pltpu.SemaphoreType.REGULAR,
        ],
    )(ktf, vtf)


def _one_head(qT, klT, vlT, krT, vrT):
    tn = (((0,), (0,)), ((), ()))
    kk = (((1,), (1,)), ((), ()))
    sl = lax.dot_general(qT, klT, tn, preferred_element_type=jnp.float32) * SCALE
    sr = lax.dot_general(qT, krT, tn, preferred_element_type=jnp.float32) * SCALE
    m = jnp.maximum(jnp.max(sl, axis=1, keepdims=True),
                    jnp.max(sr, axis=1, keepdims=True))
    pl_ = jnp.exp(sl - m)
    pr = jnp.exp(sr - m)
    denom = jnp.sum(pl_, axis=1, keepdims=True) + jnp.sum(pr, axis=1, keepdims=True)
    pl_ = (pl_ / denom).astype(jnp.bfloat16)
    pr = (pr / denom).astype(jnp.bfloat16)
    ol = lax.dot_general(vlT, pl_, kk, preferred_element_type=jnp.float32)
    orr = lax.dot_general(vrT, pr, kk, preferred_element_type=jnp.float32)
    return ol + orr


def _attn_body(q_ref, kl_ref, vl_ref, kr_ref, vr_ref, o_ref):
    for h in range(H):
        o_ref[0, h] = _one_head(
            q_ref[0, h].astype(jnp.bfloat16),
            kl_ref[0, h].astype(jnp.bfloat16),
            vl_ref[0, h].astype(jnp.bfloat16),
            kr_ref[0, h],
            vr_ref[0, h],
        )


def kernel(Q, K, V):
    QT = jnp.transpose(Q, (0, 2, 3, 1))
    KT = jnp.transpose(K, (0, 2, 3, 1))
    VT = jnp.transpose(V, (0, 2, 3, 1))

    k_rem, v_rem = _exchange(KT.reshape(ROWS, SQ), VT.reshape(ROWS, SQ))
    kr4 = k_rem.reshape(B, H, D, SQ)
    vr4 = v_rem.reshape(B, H, D, SQ)

    blk = lambda: pl.BlockSpec((1, H, D, SQ), lambda b: (b, 0, 0, 0))
    out = pl.pallas_call(
        _attn_body,
        grid=(B,),
        in_specs=[blk() for _ in range(5)],
        out_specs=blk(),
        out_shape=jax.ShapeDtypeStruct((B, H, D, SQ), jnp.float32),
        compiler_params=pltpu.CompilerParams(
            dimension_semantics=("arbitrary",)),
    )(QT, KT, VT, kr4, vr4)
    return out.transpose(0, 3, 1, 2)
